# baseline (speedup 1.0000x reference)
import functools

import jax
import jax.numpy as jnp
from jax import lax
from jax.experimental import pallas as pl
from jax.experimental.pallas import tpu as pltpu
from jax.experimental.pallas import tpu_sc as plsc

VOCAB = 1000
B, T = 1024, 50
NTOK = B * T
NC, NS = 2, 16
NW = NC * NS
BPW = NTOK // NW
CHUNK = 32
NCH = BPW // CHUNK

_mesh = plsc.VectorSubcoreMesh(core_axis_name="c", subcore_axis_name="s")


@functools.partial(
    pl.kernel,
    mesh=_mesh,
    out_type=jax.ShapeDtypeStruct((NTOK, VOCAB), jnp.float32),
    scratch_types=[
        pltpu.VMEM((BPW,), jnp.int32),
        pltpu.VMEM((CHUNK, VOCAB), jnp.float32),
        pltpu.SemaphoreType.DMA,
    ],
    compiler_params=pltpu.CompilerParams(use_tc_tiling_on_sc=False),
)
def _gather_kernel(table_hbm, idx_hbm, out_hbm, idx_v, buf, sem):
    wid = lax.axis_index("s") * NC + lax.axis_index("c")
    base = wid * BPW
    pltpu.sync_copy(idx_hbm.at[pl.ds(base, BPW)], idx_v)

    def step(ch, carry):
        off = ch * CHUNK
        pltpu.async_copy(
            table_hbm.at[idx_v.at[pl.ds(off, CHUNK)]], buf, sem
        ).wait()
        pltpu.sync_copy(buf, out_hbm.at[pl.ds(base + off, CHUNK)])
        return carry

    lax.fori_loop(0, NCH, step, 0)


def kernel(idx, table):
    flat = idx.reshape(NTOK).astype(jnp.int32)
    out = _gather_kernel(table, flat)
    return out.reshape(B, T, VOCAB)


# speedup vs baseline: 1.0478x; 1.0478x over previous
import functools

import jax
import jax.numpy as jnp
from jax import lax
from jax.experimental import pallas as pl
from jax.experimental.pallas import tpu as pltpu
from jax.experimental.pallas import tpu_sc as plsc

VOCAB = 1000
B, T = 1024, 50
NTOK = B * T
NC, NS = 2, 16
NW = NC * NS
BPW = NTOK // NW
CHUNK = 16
NCH = BPW // CHUNK
NBUF = 4

_mesh = plsc.VectorSubcoreMesh(core_axis_name="c", subcore_axis_name="s")


@functools.partial(
    pl.kernel,
    mesh=_mesh,
    out_type=jax.ShapeDtypeStruct((NTOK, VOCAB), jnp.float32),
    scratch_types=[
        pltpu.VMEM((BPW,), jnp.int32),
        [pltpu.VMEM((CHUNK, VOCAB), jnp.float32) for _ in range(NBUF)],
        [pltpu.SemaphoreType.DMA for _ in range(NBUF)],
    ],
    compiler_params=pltpu.CompilerParams(use_tc_tiling_on_sc=False),
)
def _gather_kernel(table_hbm, idx_hbm, out_hbm, idx_v, bufs, sems):
    wid = lax.axis_index("s") * NC + lax.axis_index("c")
    base = wid * BPW
    pltpu.sync_copy(idx_hbm.at[pl.ds(base, BPW)], idx_v)

    def start_gather(ch, b):
        pltpu.async_copy(
            table_hbm.at[idx_v.at[pl.ds(ch * CHUNK, CHUNK)]], bufs[b], sems[b]
        )

    for b in range(NBUF - 1):
        start_gather(b, b)

    def step(i, carry):
        for b in range(NBUF):
            ch = i * NBUF + b
            pltpu.make_async_copy(
                table_hbm.at[pl.ds(0, CHUNK)], bufs[b], sems[b]
            ).wait()
            nxt = ch + NBUF - 1

            @pl.when(nxt < NCH)
            def _():
                start_gather(nxt, (b + NBUF - 1) % NBUF)

            pltpu.sync_copy(bufs[b], out_hbm.at[pl.ds(base + ch * CHUNK, CHUNK)])
        return carry

    lax.fori_loop(0, NCH // NBUF, step, 0)


def kernel(idx, table):
    flat = idx.reshape(NTOK).astype(jnp.int32)
    out = _gather_kernel(table, flat)
    return out.reshape(B, T, VOCAB)


# speedup vs baseline: 1.1613x; 1.1083x over previous
import functools

import jax
import jax.numpy as jnp
from jax import lax
from jax.experimental import pallas as pl
from jax.experimental.pallas import tpu as pltpu
from jax.experimental.pallas import tpu_sc as plsc

VOCAB = 1000
B, T = 1024, 50
NTOK = B * T
NC, NS = 2, 16
NW = NC * NS
BPW = NTOK // NW
CHUNK = 16
NCH = BPW // CHUNK
NBUF = 4

_mesh = plsc.VectorSubcoreMesh(core_axis_name="c", subcore_axis_name="s")


@functools.partial(
    pl.kernel,
    mesh=_mesh,
    out_type=jax.ShapeDtypeStruct((NTOK, VOCAB), jnp.float32),
    scratch_types=[
        pltpu.VMEM((BPW,), jnp.int32),
        pltpu.VMEM_SHARED((VOCAB, VOCAB), jnp.float32),
        [pltpu.VMEM((CHUNK, VOCAB), jnp.float32) for _ in range(NBUF)],
        [pltpu.SemaphoreType.DMA for _ in range(NBUF)],
    ],
    compiler_params=pltpu.CompilerParams(use_tc_tiling_on_sc=False),
)
def _gather_kernel(table_hbm, idx_hbm, out_hbm, idx_v, table_sh, bufs, sems):
    sid = lax.axis_index("s")
    wid = sid * NC + lax.axis_index("c")
    base = wid * BPW
    pltpu.sync_copy(idx_hbm.at[pl.ds(base, BPW)], idx_v)

    @pl.when(sid == 0)
    def _():
        pltpu.sync_copy(table_hbm, table_sh)

    plsc.subcore_barrier()

    def start_gather(ch, b):
        pltpu.async_copy(
            table_sh.at[idx_v.at[pl.ds(ch * CHUNK, CHUNK)]], bufs[b], sems[b]
        )

    for b in range(NBUF - 1):
        start_gather(b, b)

    def step(i, carry):
        for b in range(NBUF):
            ch = i * NBUF + b
            pltpu.make_async_copy(
                table_hbm.at[pl.ds(0, CHUNK)], bufs[b], sems[b]
            ).wait()
            nxt = ch + NBUF - 1

            @pl.when(nxt < NCH)
            def _():
                start_gather(nxt, (b + NBUF - 1) % NBUF)

            pltpu.sync_copy(bufs[b], out_hbm.at[pl.ds(base + ch * CHUNK, CHUNK)])
        return carry

    lax.fori_loop(0, NCH // NBUF, step, 0)


def kernel(idx, table):
    flat = idx.reshape(NTOK).astype(jnp.int32)
    out = _gather_kernel(table, flat)
    return out.reshape(B, T, VOCAB)
